# baseline (speedup 1.0000x reference)
import functools

import jax
import jax.numpy as jnp
from jax import lax
from jax.experimental import pallas as pl
from jax.experimental.pallas import tpu as pltpu
from jax.experimental.pallas import tpu_sc as plsc

B = 4096
NUM_NUM = 13
N_CAT = 26
VOCAB = 100000
D = 64
NTOK = NUM_NUM + N_CAT

NCORES = 2
NSUB = 16
LANES = 16
NW = NCORES * NSUB
BPW = B // NW
CB = 4
NCHUNK = BPW // CB
XN_PAD = 16
NPAIR = N_CAT * VOCAB // 2

_mesh = plsc.VectorSubcoreMesh(core_axis_name="c", subcore_axis_name="s")


@functools.partial(
    pl.kernel,
    out_type=jax.ShapeDtypeStruct((B, NTOK, D), jnp.float32),
    mesh=_mesh,
    scratch_types=[
        pltpu.VMEM((BPW, N_CAT), jnp.int32),
        pltpu.VMEM((BPW, N_CAT), jnp.int32),
        pltpu.VMEM((BPW, XN_PAD), jnp.float32),
        pltpu.VMEM((NUM_NUM, D), jnp.float32),
        pltpu.VMEM((NUM_NUM, D), jnp.float32),
        pltpu.VMEM((CB, N_CAT, 2 * D), jnp.float32),
        pltpu.VMEM((CB, NTOK, D), jnp.float32),
        pltpu.SemaphoreType.DMA,
        pltpu.SemaphoreType.DMA,
    ],
)
def _tokenizer(xnum_hbm, xcat_hbm, a_hbm, b_hbm, tab_hbm, out_hbm,
               xc_v, idx_v, xn_v, a_v, b_v, pbuf, cbuf, gsem, osem):
    wid = lax.axis_index("s") * NCORES + lax.axis_index("c")
    b0 = wid * BPW

    pltpu.sync_copy(xcat_hbm.at[pl.ds(b0, BPW)], xc_v)
    pltpu.sync_copy(xnum_hbm.at[pl.ds(b0, BPW)], xn_v)
    pltpu.sync_copy(a_hbm, a_v)
    pltpu.sync_copy(b_hbm, b_v)

    off_lo = lax.iota(jnp.int32, LANES) * VOCAB
    off_hi = (lax.iota(jnp.int32, LANES) + (N_CAT - LANES)) * VOCAB
    hi_base = N_CAT - LANES

    def _idx_body(r, _):
        lo = xc_v[r, pl.ds(0, LANES)] + off_lo
        hi = xc_v[r, pl.ds(hi_base, LANES)] + off_hi
        idx_v[r, pl.ds(0, LANES)] = lax.shift_right_logical(lo, 1)
        idx_v[r, pl.ds(hi_base, LANES)] = lax.shift_right_logical(hi, 1)
        return 0

    lax.fori_loop(0, BPW, _idx_body, 0)

    def _chunk_body(c, _):
        ghs = []
        for lb in range(CB):
            ghs.append(pltpu.async_copy(
                tab_hbm.at[idx_v.at[c * CB + lb]], pbuf.at[lb], gsem))

        for lb in range(CB):
            xrow = xn_v[c * CB + lb, pl.ds(0, LANES)]
            for k in range(NUM_NUM):
                x = xrow[k]
                for ds_ in range(D // LANES):
                    cbuf[lb, k, pl.ds(ds_ * LANES, LANES)] = (
                        a_v[k, pl.ds(ds_ * LANES, LANES)]
                        + b_v[k, pl.ds(ds_ * LANES, LANES)] * x)

        for gh in ghs:
            gh.wait()

        for lb in range(CB):
            plo = xc_v[c * CB + lb, pl.ds(0, LANES)]
            phi = xc_v[c * CB + lb, pl.ds(hi_base, LANES)]
            for j in range(N_CAT):
                par = (plo[j] if j < LANES else phi[j - hi_base]) & 1
                half = par * D
                for ds_ in range(D // LANES):
                    cbuf[lb, NUM_NUM + j, pl.ds(ds_ * LANES, LANES)] = (
                        pbuf[lb, j, pl.ds(half + ds_ * LANES, LANES)])

        pltpu.async_copy(
            cbuf, out_hbm.at[pl.ds(b0 + c * CB, CB)], osem).wait()
        return 0

    lax.fori_loop(0, NCHUNK, _chunk_body, 0)


def kernel(x_num, x_cat, a, b, tables):
    xn = jnp.pad(x_num, ((0, 0), (0, XN_PAD - NUM_NUM)))
    tab = tables.reshape(NPAIR, 2 * D)
    return _tokenizer(xn, x_cat, a, b, tab)


# speedup vs baseline: 1.0125x; 1.0125x over previous
import functools

import jax
import jax.numpy as jnp
from jax import lax
from jax.experimental import pallas as pl
from jax.experimental.pallas import tpu as pltpu
from jax.experimental.pallas import tpu_sc as plsc

B = 4096
NUM_NUM = 13
N_CAT = 26
VOCAB = 100000
D = 64
NTOK = NUM_NUM + N_CAT

NCORES = 2
NSUB = 16
LANES = 16
NW = NCORES * NSUB
BPW = B // NW
CB = 4
NCHUNK = BPW // CB
NBUF = 2
IDXW = CB * N_CAT
XN_PAD = 16
NPAIR = N_CAT * VOCAB // 2
HI0 = N_CAT - LANES

_mesh = plsc.VectorSubcoreMesh(core_axis_name="c", subcore_axis_name="s")


@functools.partial(
    pl.kernel,
    out_type=jax.ShapeDtypeStruct((B, NTOK, D), jnp.float32),
    mesh=_mesh,
    scratch_types=[
        pltpu.VMEM((BPW, N_CAT), jnp.int32),
        pltpu.VMEM((BPW, XN_PAD), jnp.float32),
        pltpu.VMEM((NUM_NUM, D), jnp.float32),
        pltpu.VMEM((NUM_NUM, D), jnp.float32),
        pltpu.VMEM((NCHUNK, IDXW), jnp.int32),
        pltpu.VMEM((IDXW, 2 * D), jnp.float32),
        pltpu.VMEM((IDXW, 2 * D), jnp.float32),
        pltpu.VMEM((CB, NTOK, D), jnp.float32),
        pltpu.VMEM((CB, NTOK, D), jnp.float32),
        pltpu.SemaphoreType.DMA,
        pltpu.SemaphoreType.DMA,
        pltpu.SemaphoreType.DMA,
        pltpu.SemaphoreType.DMA,
    ],
)
def _tokenizer(xnum_hbm, xcat_hbm, a_hbm, b_hbm, tab_hbm, out_hbm,
               xc_v, xn_v, a_v, b_v, idx_v,
               pbuf0, pbuf1, cbuf0, cbuf1, gsem0, gsem1, osem0, osem1):
    pbufs = [pbuf0, pbuf1]
    cbufs = [cbuf0, cbuf1]
    gsems = [gsem0, gsem1]
    osems = [osem0, osem1]

    wid = lax.axis_index("s") * NCORES + lax.axis_index("c")
    b0 = wid * BPW

    pltpu.sync_copy(xcat_hbm.at[pl.ds(b0, BPW)], xc_v)
    pltpu.sync_copy(xnum_hbm.at[pl.ds(b0, BPW)], xn_v)
    pltpu.sync_copy(a_hbm, a_v)
    pltpu.sync_copy(b_hbm, b_v)

    off_lo = lax.iota(jnp.int32, LANES) * VOCAB
    off_hi = (lax.iota(jnp.int32, LANES) + HI0) * VOCAB

    def _idx_body(c, _):
        for lb in range(CB):
            r = c * CB + lb
            lo = xc_v[r, pl.ds(0, LANES)] + off_lo
            hi = xc_v[r, pl.ds(HI0, LANES)] + off_hi
            idx_v[c, pl.ds(lb * N_CAT, LANES)] = lax.shift_right_logical(lo, 1)
            idx_v[c, pl.ds(lb * N_CAT + HI0, LANES)] = (
                lax.shift_right_logical(hi, 1))
        return 0

    lax.fori_loop(0, NCHUNK, _idx_body, 0)

    for s in range(NBUF):
        pltpu.async_copy(tab_hbm.at[idx_v.at[s]], pbufs[s], gsems[s])

    def _super_body(g, _):
        for s in range(NBUF):
            c = g * NBUF + s

            @pl.when(c >= NBUF)
            def _():
                pltpu.make_async_copy(
                    cbufs[s],
                    out_hbm.at[pl.ds(b0 + (c - NBUF) * CB, CB)],
                    osems[s]).wait()

            for lb in range(CB):
                xrow = xn_v[c * CB + lb, pl.ds(0, LANES)]
                for k in range(NUM_NUM):
                    x = xrow[k]
                    for d0 in range(D // LANES):
                        cbufs[s][lb, k, pl.ds(d0 * LANES, LANES)] = (
                            a_v[k, pl.ds(d0 * LANES, LANES)]
                            + b_v[k, pl.ds(d0 * LANES, LANES)] * x)

            pltpu.make_async_copy(
                tab_hbm.at[idx_v.at[c]], pbufs[s], gsems[s]).wait()

            for lb in range(CB):
                plo = xc_v[c * CB + lb, pl.ds(0, LANES)]
                phi = xc_v[c * CB + lb, pl.ds(HI0, LANES)]
                for j in range(N_CAT):
                    par = (plo[j] if j < LANES else phi[j - HI0]) & 1
                    half = par * D
                    for d0 in range(D // LANES):
                        cbufs[s][lb, NUM_NUM + j,
                                 pl.ds(d0 * LANES, LANES)] = (
                            pbufs[s][lb * N_CAT + j,
                                     pl.ds(half + d0 * LANES, LANES)])

            nxt = lax.rem(c + NBUF, NCHUNK)
            pltpu.async_copy(tab_hbm.at[idx_v.at[nxt]], pbufs[s], gsems[s])

            pltpu.async_copy(
                cbufs[s], out_hbm.at[pl.ds(b0 + c * CB, CB)], osems[s])
        return 0

    lax.fori_loop(0, NCHUNK // NBUF, _super_body, 0)

    for s in range(NBUF):
        pltpu.make_async_copy(
            tab_hbm.at[idx_v.at[s]], pbufs[s], gsems[s]).wait()
        pltpu.make_async_copy(
            cbufs[s],
            out_hbm.at[pl.ds(b0 + (NCHUNK - NBUF + s) * CB, CB)],
            osems[s]).wait()


def kernel(x_num, x_cat, a, b, tables):
    xn = jnp.pad(x_num, ((0, 0), (0, XN_PAD - NUM_NUM)))
    tab = tables.reshape(NPAIR, 2 * D)
    return _tokenizer(xn, x_cat, a, b, tab)


# speedup vs baseline: 1.9746x; 1.9501x over previous
import functools

import jax
import jax.numpy as jnp
from jax import lax
from jax.experimental import pallas as pl
from jax.experimental.pallas import tpu as pltpu
from jax.experimental.pallas import tpu_sc as plsc

B = 4096
NUM_NUM = 13
N_CAT = 26
VOCAB = 100000
D = 64
NTOK = NUM_NUM + N_CAT

NCORES = 2
NSUB = 16
LANES = 16
NW = NCORES * NSUB
BPW = B // NW
NBUF = 2
SUB = 8
PROWS = N_CAT * SUB
XN_PAD = 16
NROWS = N_CAT * VOCAB
HI0 = N_CAT - LANES

_mesh = plsc.VectorSubcoreMesh(core_axis_name="c", subcore_axis_name="s")


@functools.partial(
    pl.kernel,
    out_type=jax.ShapeDtypeStruct((B, NTOK, D), jnp.float32),
    mesh=_mesh,
    scratch_types=[
        pltpu.VMEM((BPW, N_CAT), jnp.int32),
        pltpu.VMEM((BPW, XN_PAD), jnp.float32),
        pltpu.VMEM((NUM_NUM, D), jnp.float32),
        pltpu.VMEM((NUM_NUM, D), jnp.float32),
        pltpu.VMEM((PROWS, D), jnp.float32),
        pltpu.VMEM((PROWS, D), jnp.float32),
        pltpu.VMEM((1, NTOK, D), jnp.float32),
        pltpu.VMEM((1, NTOK, D), jnp.float32),
        pltpu.SemaphoreType.DMA,
        pltpu.SemaphoreType.DMA,
        pltpu.SemaphoreType.DMA,
        pltpu.SemaphoreType.DMA,
    ],
)
def _tokenizer(xnum_hbm, xcat_hbm, a_hbm, b_hbm, tab_hbm, out_hbm,
               xc_v, xn_v, a_v, b_v,
               pbuf0, pbuf1, cbuf0, cbuf1, gsem0, gsem1, osem0, osem1):
    pbufs = [pbuf0, pbuf1]
    cbufs = [cbuf0, cbuf1]
    gsems = [gsem0, gsem1]
    osems = [osem0, osem1]

    wid = lax.axis_index("s") * NCORES + lax.axis_index("c")
    b0 = wid * BPW

    pltpu.sync_copy(xcat_hbm.at[pl.ds(b0, BPW)], xc_v)
    pltpu.sync_copy(xnum_hbm.at[pl.ds(b0, BPW)], xn_v)
    pltpu.sync_copy(a_hbm, a_v)
    pltpu.sync_copy(b_hbm, b_v)

    def _issue_fetches(c, s):
        plo = xc_v[c, pl.ds(0, LANES)]
        phi = xc_v[c, pl.ds(HI0, LANES)]
        for j in range(N_CAT):
            xcj = plo[j] if j < LANES else phi[j - HI0]
            g = lax.shift_right_logical(xcj + j * VOCAB, 3)
            row0 = pl.multiple_of(g * SUB, SUB)
            pltpu.async_copy(
                tab_hbm.at[pl.ds(row0, SUB)],
                pbufs[s].at[pl.ds(j * SUB, SUB)],
                gsems[s])

    def _drain_fetches(s):
        pltpu.make_async_copy(
            tab_hbm.at[pl.ds(0, PROWS)], pbufs[s], gsems[s]).wait()

    def _out_descr(c, s):
        return pltpu.make_async_copy(
            cbufs[s], out_hbm.at[pl.ds(b0 + c, 1)], osems[s])

    _issue_fetches(0, 0)

    def _super_body(g, _):
        for s in range(NBUF):
            c = g * NBUF + s
            s1 = 1 - s

            @pl.when(c >= NBUF)
            def _():
                _out_descr(c - NBUF, s).wait()

            xrow = xn_v[c, pl.ds(0, LANES)]
            for k in range(NUM_NUM):
                x = xrow[k]
                for d0 in range(D // LANES):
                    cbufs[s][0, k, pl.ds(d0 * LANES, LANES)] = (
                        a_v[k, pl.ds(d0 * LANES, LANES)]
                        + b_v[k, pl.ds(d0 * LANES, LANES)] * x)

            _drain_fetches(s)
            plo = xc_v[c, pl.ds(0, LANES)]
            phi = xc_v[c, pl.ds(HI0, LANES)]
            for j in range(N_CAT):
                sub = (plo[j] if j < LANES else phi[j - HI0]) & (SUB - 1)
                for d0 in range(D // LANES):
                    cbufs[s][0, NUM_NUM + j, pl.ds(d0 * LANES, LANES)] = (
                        pbufs[s][j * SUB + sub, pl.ds(d0 * LANES, LANES)])

            _out_descr(c, s).start()

            @pl.when(c + 1 < BPW)
            def _():
                _issue_fetches(c + 1, s1)
        return 0

    lax.fori_loop(0, BPW // NBUF, _super_body, 0)

    for s in range(NBUF):
        _out_descr(BPW - NBUF + s, s).wait()


def kernel(x_num, x_cat, a, b, tables):
    xn = jnp.pad(x_num, ((0, 0), (0, XN_PAD - NUM_NUM)))
    tab = tables.reshape(NROWS, D)
    return _tokenizer(xn, x_cat, a, b, tab)
